# baseline (speedup 1.0000x reference)
import functools

import jax
import jax.numpy as jnp
from jax import lax
from jax.experimental import pallas as pl
from jax.experimental.pallas import tpu as pltpu
from jax.experimental.pallas import tpu_sc as plsc

N_NODES = 10000
N_RELS = 200
DIM = 128
NW = 32
CHUNK = 128
CPW = 80
E_PAD = NW * CPW * CHUNK
ACC_ROWS = 10112
C_ROWS = 10080
CFLAT = C_ROWS * N_RELS
ROW_BLK = 2000

_mesh = plsc.VectorSubcoreMesh(core_axis_name="c", subcore_axis_name="s")

_Z16 = functools.partial(jnp.zeros, (16,), jnp.float32)


CHALF = CFLAT // 2
CDUMP = 16000
CLOC = CHALF + CDUMP


N_E = 320000
HGRP = 312
SCH = 64
GEDGES = 1024
TILE_E = E_PAD // NW
NGRP = TILE_E // GEDGES
NCHK = TILE_E // SCH


def _hist_body(dstp, etp, out, cacc, dstb, etb, idxb0, idxb1, ones, wb0, wb1,
               zbuf, sem0, sem1, ssem0, ssem1):
    cid = lax.axis_index("c")
    sid = lax.axis_index("s")
    base = cid * CHALF

    @pl.loop(0, 500)
    def _zfill(i):
        zbuf[pl.ds(i * 16, 16)] = _Z16()

    @pl.loop(0, 8)
    def _ofill(i):
        ones[pl.ds(i * 16, 16)] = jnp.ones((16,), jnp.float32)

    @pl.loop(0, 8)
    def _zero(i):
        pltpu.sync_copy(zbuf, cacc.at[pl.ds(sid * (CLOC // 16) + i * 8000, 8000)])

    plsc.subcore_barrier()

    def chunk_idx(eoff, j, idxb):
        for k in range(CHUNK // 16):
            d = dstb[pl.ds(eoff + j * CHUNK + k * 16, 16)]
            t = etb[pl.ds(eoff + j * CHUNK + k * 16, 16)]
            loc = d * N_RELS + t - base
            valid = jnp.logical_and(loc >= 0, loc < CHALF)
            dump = CHALF + k * 16 + lax.iota(jnp.int32, 16)
            idxb[pl.ds(k * 16, 16)] = jnp.where(valid, loc, dump)

    def swait(idxb, ssem):
        pltpu.make_async_copy(ones, cacc.at[idxb], ssem).wait()

    gstart = sid * 19 + jnp.minimum(sid, 8)
    ngrp = 19 + (sid < 8).astype(jnp.int32)

    @pl.loop(0, ngrp)
    def _grp(g):
        eb = (gstart + g) * GEDGES
        pltpu.sync_copy(dstp.at[pl.ds(eb, GEDGES)], dstb)
        pltpu.sync_copy(etp.at[pl.ds(eb, GEDGES)], etb)
        for j in range(8):
            idxb = idxb0 if j % 2 == 0 else idxb1
            ssem = ssem0 if j % 2 == 0 else ssem1
            if j < 2:
                @pl.when(g > 0)
                def _w():
                    swait(idxb, ssem)
            else:
                swait(idxb, ssem)
            chunk_idx(0, j, idxb)
            pltpu.async_copy(ones, cacc.at[idxb], ssem, add=True)

    swait(idxb0, ssem0)
    swait(idxb1, ssem1)

    @pl.when(sid == 15)
    def _tail():
        pltpu.sync_copy(dstp.at[pl.ds(HGRP * GEDGES, 512)],
                        dstb.at[pl.ds(0, 512)])
        pltpu.sync_copy(etp.at[pl.ds(HGRP * GEDGES, 512)],
                        etb.at[pl.ds(0, 512)])
        for j in range(4):
            chunk_idx(0, j, idxb0)
            pltpu.sync_copy(ones, cacc.at[idxb0], add=True)

    plsc.subcore_barrier()

    wout = base + sid * (CHALF // 16)
    wloc = sid * (CHALF // 16)

    def _piece(i, buf, sem):
        pltpu.sync_copy(cacc.at[pl.ds(wloc + i * 9000, 9000)], buf)
        return pltpu.async_copy(buf, out.at[pl.ds(wout + i * 9000, 9000)], sem)

    d0 = _piece(0, wb0, sem0)

    @pl.loop(0, 3)
    def _wo(i):
        d1 = _piece(2 * i + 1, wb1, sem1)
        pltpu.make_async_copy(wb0, out.at[pl.ds(wout, 9000)], sem0).wait()

        @pl.when(i < 2)
        def _more():
            _piece(2 * i + 2, wb0, sem0)

        d1.wait()

    _piece(6, wb0, sem0).wait()


_hist_call = functools.partial(
    pl.kernel,
    _hist_body,
    out_type=jax.ShapeDtypeStruct((CFLAT,), jnp.float32),
    mesh=_mesh,
    scratch_types=[
        pltpu.VMEM_SHARED((CLOC,), jnp.float32),
        pltpu.VMEM((GEDGES,), jnp.int32),
        pltpu.VMEM((GEDGES,), jnp.int32),
        pltpu.VMEM((CHUNK,), jnp.int32),
        pltpu.VMEM((CHUNK,), jnp.int32),
        pltpu.VMEM((CHUNK,), jnp.float32),
        pltpu.VMEM((9000,), jnp.float32),
        pltpu.VMEM((9000,), jnp.float32),
        pltpu.VMEM((8000,), jnp.float32),
        pltpu.SemaphoreType.DMA,
        pltpu.SemaphoreType.DMA,
        pltpu.SemaphoreType.DMA,
        pltpu.SemaphoreType.DMA,
    ],
)()


def _spmm_body(table, srcp, dstp, out, acc, srcba, dstba, srcbb, dstbb,
               rows, dsaves, gsems, ssems, isem):
    cid = lax.axis_index("c")
    sid = lax.axis_index("s")
    wid = cid * 16 + sid
    myrow = sid * (ACC_ROWS // 16)
    ebase = wid * TILE_E

    @pl.loop(0, SCH)
    def _zfill(i):
        for k in range(DIM // 16):
            rows[0][i, pl.ds(k * 16, 16)] = _Z16()

    @pl.loop(0, 9)
    def _zero(i):
        pltpu.sync_copy(rows[0], acc.at[pl.ds(myrow + i * SCH, SCH)])

    pltpu.sync_copy(rows[0].at[pl.ds(0, 56)], acc.at[pl.ds(myrow + 576, 56)])
    plsc.subcore_barrier()

    def stage(gidx, sb, db, sync):
        s = pltpu.async_copy(srcp.at[pl.ds(ebase + gidx * GEDGES, GEDGES)],
                             sb, isem)
        d = pltpu.async_copy(dstp.at[pl.ds(ebase + gidx * GEDGES, GEDGES)],
                             db, isem)
        if sync:
            s.wait()
            d.wait()

    def wait_stage(sb, db):
        pltpu.make_async_copy(srcp.at[pl.ds(0, GEDGES)], sb, isem).wait()
        pltpu.make_async_copy(dstp.at[pl.ds(0, GEDGES)], db, isem).wait()

    def gissue(b, sb, off):
        pltpu.async_copy(table.at[sb.at[pl.ds(off * SCH, SCH)]],
                         rows[b], gsems[b])

    def gwait(b, sb):
        pltpu.make_async_copy(table.at[sb.at[pl.ds(0, SCH)]],
                              rows[b], gsems[b]).wait()

    def sissue(b):
        pltpu.async_copy(rows[b], acc.at[dsaves[b]], ssems[b], add=True)

    def swait(b):
        pltpu.make_async_copy(rows[b], acc.at[dsaves[b]], ssems[b]).wait()

    stage(0, srcba, dstba, True)
    stage(1, srcbb, dstbb, False)
    for c in range(3):
        gissue(c, srcba, c)

    last = NCHK // 32 - 1

    @pl.loop(0, NCHK // 32)
    def _edges(i):
        for local in range(32):
            b = local % 4
            sb, db = (srcba, dstba) if local < 16 else (srcbb, dstbb)
            off = local % 16
            gwait(b, sb)
            for k in range(SCH // 16):
                dsaves[b][pl.ds(k * 16, 16)] = db[pl.ds(off * SCH + k * 16, 16)]
            sissue(b)
            pb = (b + 3) % 4
            if local == 0:
                @pl.when(i > 0)
                def _w0():
                    swait(pb)

                gissue(pb, sb, off + 3)
            elif local == 13:
                swait(pb)
                wait_stage(srcbb, dstbb)
                gissue(pb, srcbb, 0)
            elif local < 13:
                swait(pb)
                gissue(pb, sb, off + 3)
            elif local < 29:
                swait(pb)
                gissue(pb, srcbb, local - 13)
                if local == 15:
                    @pl.when(i < last)
                    def _ra():
                        stage(2 * i + 2, srcba, dstba, False)
            elif local == 29:
                @pl.when(i < last)
                def _l29():
                    swait(pb)
                    wait_stage(srcba, dstba)
                    gissue(pb, srcba, 0)
            else:
                @pl.when(i < last)
                def _gt():
                    swait(pb)
                    gissue(pb, srcba, local - 29)

                if local == 31:
                    @pl.when(i < last)
                    def _rb():
                        stage(2 * i + 3, srcbb, dstbb, False)

    for b in range(4):
        swait(b)
    plsc.subcore_barrier()

    def _wo(b, p, n):
        return pltpu.make_async_copy(
            rows[b].at[pl.ds(0, n)],
            out.at[cid, pl.ds(myrow + p * SCH, n)], ssems[b])

    sizes = [SCH] * 9 + [56]
    for p, n in enumerate(sizes):
        b = p % 4
        if p >= 4:
            _wo(b, p - 4, sizes[p - 4]).wait()
        pltpu.sync_copy(acc.at[pl.ds(myrow + p * SCH, n)],
                        rows[b].at[pl.ds(0, n)])
        pltpu.async_copy(rows[b].at[pl.ds(0, n)],
                         out.at[cid, pl.ds(myrow + p * SCH, n)], ssems[b])
    for p in range(6, 10):
        _wo(p % 4, p, sizes[p]).wait()


def _spmm_call(table, srcp, dstp):
    return pl.kernel(
        _spmm_body,
        out_type=jax.ShapeDtypeStruct((2, ACC_ROWS, DIM), jnp.float32),
        mesh=_mesh,
        scratch_types=[
            pltpu.VMEM_SHARED((ACC_ROWS, DIM), jnp.float32),
            pltpu.VMEM((GEDGES,), jnp.int32),
            pltpu.VMEM((GEDGES,), jnp.int32),
            pltpu.VMEM((GEDGES,), jnp.int32),
            pltpu.VMEM((GEDGES,), jnp.int32),
            [pltpu.VMEM((SCH, DIM), jnp.float32)] * 4,
            [pltpu.VMEM((SCH,), jnp.int32)] * 4,
            [pltpu.SemaphoreType.DMA] * 4,
            [pltpu.SemaphoreType.DMA] * 4,
            pltpu.SemaphoreType.DMA,
        ],
    )(table, srcp, dstp)


def _gather_body(x2, sub, r2, rel, out_sub, out_rel, idxb, rows, sem):
    cid = lax.axis_index("c")
    sid = lax.axis_index("s")
    wid = cid * 16 + sid
    base = wid * 32

    pltpu.sync_copy(sub.at[pl.ds(base, 32)], idxb)
    pltpu.async_copy(x2.at[idxb], rows, sem).wait()
    pltpu.sync_copy(rows, out_sub.at[pl.ds(base, 32)])

    pltpu.sync_copy(rel.at[pl.ds(base, 32)], idxb)
    pltpu.async_copy(r2.at[idxb], rows, sem).wait()
    pltpu.sync_copy(rows, out_rel.at[pl.ds(base, 32)])


def _gather_call(x2, sub, r2, rel):
    return pl.kernel(
        _gather_body,
        out_type=(jax.ShapeDtypeStruct((1024, DIM), jnp.float32),
                  jax.ShapeDtypeStruct((1024, DIM), jnp.float32)),
        mesh=_mesh,
        scratch_types=[
            pltpu.VMEM((32,), jnp.int32),
            pltpu.VMEM((32, DIM), jnp.float32),
            pltpu.SemaphoreType.DMA,
        ],
    )(x2, sub, r2, rel)


def _layer_common(a_ref, c_ref, x_ref, r_ref, w_ref, wl_ref, o_ref):
    a = a_ref[0] + a_ref[1]
    c = c_ref[...]
    deg = jnp.sum(c, axis=1)
    agg = a - jnp.dot(c, r_ref[...], preferred_element_type=jnp.float32)
    z = jnp.dot(agg, w_ref[...], preferred_element_type=jnp.float32)
    inv = 1.0 / jnp.maximum(deg, 1.0)
    loop = jnp.dot(x_ref[...], wl_ref[...], preferred_element_type=jnp.float32)
    o_ref[...] = jnp.tanh(z * inv[:, None] + loop)


_LAYER_SPECS = [
    pl.BlockSpec((2, ROW_BLK, DIM), lambda i: (0, i, 0)),
    pl.BlockSpec((ROW_BLK, N_RELS), lambda i: (i, 0)),
    pl.BlockSpec((ROW_BLK, DIM), lambda i: (i, 0)),
    pl.BlockSpec((N_RELS, DIM), lambda i: (0, 0)),
    pl.BlockSpec((DIM, DIM), lambda i: (0, 0)),
    pl.BlockSpec((DIM, DIM), lambda i: (0, 0)),
]
_REL_SPEC = pl.BlockSpec((N_RELS, DIM), lambda i: (0, 0))


def _layer1_body(a_ref, c_ref, x_ref, r_ref, w_ref, wl_ref, wr1_ref, wr2_ref,
                 o_ref, r1_ref, r2_ref):
    _layer_common(a_ref, c_ref, x_ref, r_ref, w_ref, wl_ref, o_ref)

    @pl.when(pl.program_id(0) == 0)
    def _rels():
        r1 = jnp.dot(r_ref[...], wr1_ref[...],
                     preferred_element_type=jnp.float32)
        r1_ref[...] = r1
        r2_ref[...] = jnp.dot(r1, wr2_ref[...],
                              preferred_element_type=jnp.float32)


def _layer1_call(acc, c, x, r, w, w_loop, wr1, wr2):
    return pl.pallas_call(
        _layer1_body,
        grid=(N_NODES // ROW_BLK,),
        in_specs=_LAYER_SPECS + [pl.BlockSpec((DIM, DIM), lambda i: (0, 0)),
                                 pl.BlockSpec((DIM, DIM), lambda i: (0, 0))],
        out_specs=(pl.BlockSpec((ROW_BLK, DIM), lambda i: (i, 0)),
                   _REL_SPEC, _REL_SPEC),
        out_shape=(jax.ShapeDtypeStruct((N_NODES, DIM), jnp.float32),
                   jax.ShapeDtypeStruct((N_RELS, DIM), jnp.float32),
                   jax.ShapeDtypeStruct((N_RELS, DIM), jnp.float32)),
    )(acc, c, x, r, w, w_loop, wr1, wr2)


def _emb_body(x_ref, sub_ref, rel_ref, r2_ref, sub_out, rel_out):
    i = pl.program_id(0)
    iota = lax.broadcasted_iota(jnp.int32, (1024, ROW_BLK), 1) + i * ROW_BLK
    oh = (sub_ref[...] == iota).astype(jnp.float32)
    part = jnp.dot(oh, x_ref[...], preferred_element_type=jnp.float32)

    @pl.when(i == 0)
    def _init():
        sub_out[...] = part
        iota_r = lax.broadcasted_iota(jnp.int32, (1024, N_RELS), 1)
        ohr = (rel_ref[...] == iota_r).astype(jnp.float32)
        rel_out[...] = jnp.dot(ohr, r2_ref[...],
                               preferred_element_type=jnp.float32)

    @pl.when(i > 0)
    def _acc():
        sub_out[...] += part


def _emb_call(x2, sub1, rel1, r2):
    cst = pl.BlockSpec((1024, 1), lambda i: (0, 0))
    return pl.pallas_call(
        _emb_body,
        grid=(N_NODES // ROW_BLK,),
        in_specs=[pl.BlockSpec((ROW_BLK, DIM), lambda i: (i, 0)),
                  cst, cst, _REL_SPEC],
        out_specs=(pl.BlockSpec((1024, DIM), lambda i: (0, 0)),
                   pl.BlockSpec((1024, DIM), lambda i: (0, 0))),
        out_shape=(jax.ShapeDtypeStruct((1024, DIM), jnp.float32),
                   jax.ShapeDtypeStruct((1024, DIM), jnp.float32)),
    )(x2, sub1, rel1, r2)


def _layer2_call(acc, c, x, r, w, w_loop):
    return pl.pallas_call(
        _layer_common,
        grid=(N_NODES // ROW_BLK,),
        in_specs=_LAYER_SPECS,
        out_specs=pl.BlockSpec((ROW_BLK, DIM), lambda i: (i, 0)),
        out_shape=jax.ShapeDtypeStruct((N_NODES, DIM), jnp.float32),
    )(acc, c, x, r, w, w_loop)


def kernel(edge_index, edge_type, sub, rel, init_embed, init_rel,
           W1, W1_loop, Wrel1, W2, W2_loop, Wrel2):
    src = edge_index[0]
    dst = edge_index[1]
    e = src.shape[0]
    pad = E_PAD - e
    ar = jnp.arange(pad, dtype=jnp.int32)
    src_b, dst_b = lax.optimization_barrier((src, dst))
    srcp = jnp.concatenate([src_b, (ar * 29) % N_NODES])
    dstp = jnp.concatenate([dst_b, N_NODES + (ar % 16)])

    cf = _hist_call(dst, edge_type)
    c = cf.reshape(C_ROWS, N_RELS)

    a1 = _spmm_call(init_embed, srcp, dstp)
    x1, r1, r2 = _layer1_call(a1, c, init_embed, init_rel,
                              W1, W1_loop, Wrel1, Wrel2)
    a2 = _spmm_call(x1, srcp, dstp)
    x2 = _layer2_call(a2, c, x1, r1, W2, W2_loop)

    sub_emb, rel_emb = _emb_call(x2, sub.reshape(1024, 1),
                                 rel.reshape(1024, 1), r2)
    return (sub_emb, rel_emb, x2, r2)


# speedup vs baseline: 1.0153x; 1.0153x over previous
import functools

import jax
import jax.numpy as jnp
from jax import lax
from jax.experimental import pallas as pl
from jax.experimental.pallas import tpu as pltpu
from jax.experimental.pallas import tpu_sc as plsc

N_NODES = 10000
N_RELS = 200
DIM = 128
NW = 32
CHUNK = 128
CPW = 80
E_PAD = NW * CPW * CHUNK
ACC_ROWS = 10112
C_ROWS = 10080
CFLAT = C_ROWS * N_RELS
ROW_BLK = 2000

_mesh = plsc.VectorSubcoreMesh(core_axis_name="c", subcore_axis_name="s")

_Z16 = functools.partial(jnp.zeros, (16,), jnp.float32)


CHALF = CFLAT // 2
CDUMP = 16000
CLOC = CHALF + CDUMP


N_E = 320000
HGRP = 312
SCH = 64
GEDGES = 1024
TILE_E = E_PAD // NW
NGRP = TILE_E // GEDGES
NCHK = TILE_E // SCH


def _hist_body(dstp, etp, out, cacc, dstb, etb, idxb0, idxb1, ones, wb0, wb1,
               zbuf, sem0, sem1, ssem0, ssem1):
    cid = lax.axis_index("c")
    sid = lax.axis_index("s")
    base = cid * CHALF

    @pl.loop(0, 500)
    def _zfill(i):
        zbuf[pl.ds(i * 16, 16)] = _Z16()

    @pl.loop(0, 8)
    def _ofill(i):
        ones[pl.ds(i * 16, 16)] = jnp.ones((16,), jnp.float32)

    @pl.loop(0, 8)
    def _zero(i):
        pltpu.sync_copy(zbuf, cacc.at[pl.ds(sid * (CLOC // 16) + i * 8000, 8000)])

    plsc.subcore_barrier()

    def chunk_idx(eoff, j, idxb):
        for k in range(CHUNK // 16):
            d = dstb[pl.ds(eoff + j * CHUNK + k * 16, 16)]
            t = etb[pl.ds(eoff + j * CHUNK + k * 16, 16)]
            loc = d * N_RELS + t - base
            valid = jnp.logical_and(loc >= 0, loc < CHALF)
            dump = CHALF + k * 16 + lax.iota(jnp.int32, 16)
            idxb[pl.ds(k * 16, 16)] = jnp.where(valid, loc, dump)

    def swait(idxb, ssem):
        pltpu.make_async_copy(ones, cacc.at[idxb], ssem).wait()

    gstart = sid * 19 + jnp.minimum(sid, 8)
    ngrp = 19 + (sid < 8).astype(jnp.int32)

    @pl.loop(0, ngrp)
    def _grp(g):
        eb = (gstart + g) * GEDGES
        pltpu.sync_copy(dstp.at[pl.ds(eb, GEDGES)], dstb)
        pltpu.sync_copy(etp.at[pl.ds(eb, GEDGES)], etb)
        for j in range(8):
            idxb = idxb0 if j % 2 == 0 else idxb1
            ssem = ssem0 if j % 2 == 0 else ssem1
            if j < 2:
                @pl.when(g > 0)
                def _w():
                    swait(idxb, ssem)
            else:
                swait(idxb, ssem)
            chunk_idx(0, j, idxb)
            pltpu.async_copy(ones, cacc.at[idxb], ssem, add=True)

    swait(idxb0, ssem0)
    swait(idxb1, ssem1)

    @pl.when(sid == 15)
    def _tail():
        pltpu.sync_copy(dstp.at[pl.ds(HGRP * GEDGES, 512)],
                        dstb.at[pl.ds(0, 512)])
        pltpu.sync_copy(etp.at[pl.ds(HGRP * GEDGES, 512)],
                        etb.at[pl.ds(0, 512)])
        for j in range(4):
            chunk_idx(0, j, idxb0)
            pltpu.sync_copy(ones, cacc.at[idxb0], add=True)

    plsc.subcore_barrier()

    wout = base + sid * (CHALF // 16)
    wloc = sid * (CHALF // 16)

    def _piece(i, buf, sem):
        pltpu.sync_copy(cacc.at[pl.ds(wloc + i * 9000, 9000)], buf)
        return pltpu.async_copy(buf, out.at[pl.ds(wout + i * 9000, 9000)], sem)

    d0 = _piece(0, wb0, sem0)

    @pl.loop(0, 3)
    def _wo(i):
        d1 = _piece(2 * i + 1, wb1, sem1)
        pltpu.make_async_copy(wb0, out.at[pl.ds(wout, 9000)], sem0).wait()

        @pl.when(i < 2)
        def _more():
            _piece(2 * i + 2, wb0, sem0)

        d1.wait()

    _piece(6, wb0, sem0).wait()


_hist_call = functools.partial(
    pl.kernel,
    _hist_body,
    out_type=jax.ShapeDtypeStruct((CFLAT,), jnp.float32),
    mesh=_mesh,
    scratch_types=[
        pltpu.VMEM_SHARED((CLOC,), jnp.float32),
        pltpu.VMEM((GEDGES,), jnp.int32),
        pltpu.VMEM((GEDGES,), jnp.int32),
        pltpu.VMEM((CHUNK,), jnp.int32),
        pltpu.VMEM((CHUNK,), jnp.int32),
        pltpu.VMEM((CHUNK,), jnp.float32),
        pltpu.VMEM((9000,), jnp.float32),
        pltpu.VMEM((9000,), jnp.float32),
        pltpu.VMEM((8000,), jnp.float32),
        pltpu.SemaphoreType.DMA,
        pltpu.SemaphoreType.DMA,
        pltpu.SemaphoreType.DMA,
        pltpu.SemaphoreType.DMA,
    ],
)()


def _spmm_body(table, srcp, dstp, out, acc, srcba, dstba, srcbb, dstbb,
               rows, dsaves, gsems, ssems, isem):
    cid = lax.axis_index("c")
    sid = lax.axis_index("s")
    wid = cid * 16 + sid
    myrow = sid * (ACC_ROWS // 16)
    ebase = wid * TILE_E

    @pl.loop(0, SCH)
    def _zfill(i):
        for k in range(DIM // 16):
            rows[0][i, pl.ds(k * 16, 16)] = _Z16()

    @pl.loop(0, 9)
    def _zero(i):
        pltpu.sync_copy(rows[0], acc.at[pl.ds(myrow + i * SCH, SCH)])

    pltpu.sync_copy(rows[0].at[pl.ds(0, 56)], acc.at[pl.ds(myrow + 576, 56)])
    plsc.subcore_barrier()

    def stage(gidx, sb, db, sync):
        s = pltpu.async_copy(srcp.at[pl.ds(ebase + gidx * GEDGES, GEDGES)],
                             sb, isem)
        d = pltpu.async_copy(dstp.at[pl.ds(ebase + gidx * GEDGES, GEDGES)],
                             db, isem)
        if sync:
            s.wait()
            d.wait()

    def wait_stage(sb, db):
        pltpu.make_async_copy(srcp.at[pl.ds(0, GEDGES)], sb, isem).wait()
        pltpu.make_async_copy(dstp.at[pl.ds(0, GEDGES)], db, isem).wait()

    def gissue(b, sb, off):
        pltpu.async_copy(table.at[sb.at[pl.ds(off * SCH, SCH)]],
                         rows[b], gsems[b])

    def gwait(b, sb):
        pltpu.make_async_copy(table.at[sb.at[pl.ds(0, SCH)]],
                              rows[b], gsems[b]).wait()

    def sissue(b):
        pltpu.async_copy(rows[b], acc.at[dsaves[b]], ssems[b], add=True)

    def swait(b):
        pltpu.make_async_copy(rows[b], acc.at[dsaves[b]], ssems[b]).wait()

    stage(0, srcba, dstba, True)
    stage(1, srcbb, dstbb, False)
    for c in range(3):
        gissue(c, srcba, c)

    last = NCHK // 32 - 1

    @pl.loop(0, NCHK // 32)
    def _edges(i):
        for local in range(32):
            b = local % 4
            sb, db = (srcba, dstba) if local < 16 else (srcbb, dstbb)
            off = local % 16
            gwait(b, sb)
            for k in range(SCH // 16):
                dsaves[b][pl.ds(k * 16, 16)] = db[pl.ds(off * SCH + k * 16, 16)]
            sissue(b)
            pb = (b + 3) % 4
            if local == 0:
                @pl.when(i > 0)
                def _w0():
                    swait(pb)

                gissue(pb, sb, off + 3)
            elif local == 13:
                swait(pb)
                wait_stage(srcbb, dstbb)
                gissue(pb, srcbb, 0)
            elif local < 13:
                swait(pb)
                gissue(pb, sb, off + 3)
            elif local < 29:
                swait(pb)
                gissue(pb, srcbb, local - 13)
                if local == 15:
                    @pl.when(i < last)
                    def _ra():
                        stage(2 * i + 2, srcba, dstba, False)
            elif local == 29:
                @pl.when(i < last)
                def _l29():
                    swait(pb)
                    wait_stage(srcba, dstba)
                    gissue(pb, srcba, 0)
            else:
                @pl.when(i < last)
                def _gt():
                    swait(pb)
                    gissue(pb, srcba, local - 29)

                if local == 31:
                    @pl.when(i < last)
                    def _rb():
                        stage(2 * i + 3, srcbb, dstbb, False)

    for b in range(4):
        swait(b)
    plsc.subcore_barrier()

    def _wo(b, p, n):
        return pltpu.make_async_copy(
            rows[b].at[pl.ds(0, n)],
            out.at[cid, pl.ds(myrow + p * SCH, n)], ssems[b])

    sizes = [SCH] * 9 + [56]
    for p, n in enumerate(sizes):
        b = p % 4
        if p >= 4:
            _wo(b, p - 4, sizes[p - 4]).wait()
        pltpu.sync_copy(acc.at[pl.ds(myrow + p * SCH, n)],
                        rows[b].at[pl.ds(0, n)])
        pltpu.async_copy(rows[b].at[pl.ds(0, n)],
                         out.at[cid, pl.ds(myrow + p * SCH, n)], ssems[b])
    for p in range(6, 10):
        _wo(p % 4, p, sizes[p]).wait()


def _spmm_call(table, srcp, dstp):
    return pl.kernel(
        _spmm_body,
        out_type=jax.ShapeDtypeStruct((2, ACC_ROWS, DIM), jnp.float32),
        mesh=_mesh,
        scratch_types=[
            pltpu.VMEM_SHARED((ACC_ROWS, DIM), jnp.float32),
            pltpu.VMEM((GEDGES,), jnp.int32),
            pltpu.VMEM((GEDGES,), jnp.int32),
            pltpu.VMEM((GEDGES,), jnp.int32),
            pltpu.VMEM((GEDGES,), jnp.int32),
            [pltpu.VMEM((SCH, DIM), jnp.float32)] * 4,
            [pltpu.VMEM((SCH,), jnp.int32)] * 4,
            [pltpu.SemaphoreType.DMA] * 4,
            [pltpu.SemaphoreType.DMA] * 4,
            pltpu.SemaphoreType.DMA,
        ],
    )(table, srcp, dstp)


def _gather_body(x2, sub, r2, rel, out_sub, out_rel, idxb, rows, sem):
    cid = lax.axis_index("c")
    sid = lax.axis_index("s")
    wid = cid * 16 + sid
    base = wid * 32

    pltpu.sync_copy(sub.at[pl.ds(base, 32)], idxb)
    pltpu.async_copy(x2.at[idxb], rows, sem).wait()
    pltpu.sync_copy(rows, out_sub.at[pl.ds(base, 32)])

    pltpu.sync_copy(rel.at[pl.ds(base, 32)], idxb)
    pltpu.async_copy(r2.at[idxb], rows, sem).wait()
    pltpu.sync_copy(rows, out_rel.at[pl.ds(base, 32)])


def _gather_call(x2, sub, r2, rel):
    return pl.kernel(
        _gather_body,
        out_type=(jax.ShapeDtypeStruct((1024, DIM), jnp.float32),
                  jax.ShapeDtypeStruct((1024, DIM), jnp.float32)),
        mesh=_mesh,
        scratch_types=[
            pltpu.VMEM((32,), jnp.int32),
            pltpu.VMEM((32, DIM), jnp.float32),
            pltpu.SemaphoreType.DMA,
        ],
    )(x2, sub, r2, rel)


def _layer_common(a_ref, c_ref, x_ref, r_ref, w_ref, wl_ref, o_ref):
    a = a_ref[0] + a_ref[1]
    c = c_ref[...]
    deg = jnp.sum(c, axis=1)
    agg = a - jnp.dot(c, r_ref[...], preferred_element_type=jnp.float32)
    z = jnp.dot(agg, w_ref[...], preferred_element_type=jnp.float32)
    inv = 1.0 / jnp.maximum(deg, 1.0)
    loop = jnp.dot(x_ref[...], wl_ref[...], preferred_element_type=jnp.float32)
    o_ref[...] = jnp.tanh(z * inv[:, None] + loop)


_LAYER_SPECS = [
    pl.BlockSpec((2, ROW_BLK, DIM), lambda i: (0, i, 0)),
    pl.BlockSpec((ROW_BLK, N_RELS), lambda i: (i, 0)),
    pl.BlockSpec((ROW_BLK, DIM), lambda i: (i, 0)),
    pl.BlockSpec((N_RELS, DIM), lambda i: (0, 0)),
    pl.BlockSpec((DIM, DIM), lambda i: (0, 0)),
    pl.BlockSpec((DIM, DIM), lambda i: (0, 0)),
]
_REL_SPEC = pl.BlockSpec((N_RELS, DIM), lambda i: (0, 0))


def _layer1_body(a_ref, c_ref, x_ref, r_ref, w_ref, wl_ref, wr1_ref, wr2_ref,
                 o_ref, r1_ref, r2_ref):
    _layer_common(a_ref, c_ref, x_ref, r_ref, w_ref, wl_ref, o_ref)

    @pl.when(pl.program_id(0) == 0)
    def _rels():
        r1 = jnp.dot(r_ref[...], wr1_ref[...],
                     preferred_element_type=jnp.float32)
        r1_ref[...] = r1
        r2_ref[...] = jnp.dot(r1, wr2_ref[...],
                              preferred_element_type=jnp.float32)


def _layer1_call(acc, c, x, r, w, w_loop, wr1, wr2):
    return pl.pallas_call(
        _layer1_body,
        grid=(N_NODES // ROW_BLK,),
        in_specs=_LAYER_SPECS + [pl.BlockSpec((DIM, DIM), lambda i: (0, 0)),
                                 pl.BlockSpec((DIM, DIM), lambda i: (0, 0))],
        out_specs=(pl.BlockSpec((ROW_BLK, DIM), lambda i: (i, 0)),
                   _REL_SPEC, _REL_SPEC),
        out_shape=(jax.ShapeDtypeStruct((N_NODES, DIM), jnp.float32),
                   jax.ShapeDtypeStruct((N_RELS, DIM), jnp.float32),
                   jax.ShapeDtypeStruct((N_RELS, DIM), jnp.float32)),
    )(acc, c, x, r, w, w_loop, wr1, wr2)


def _layer2_call(acc, c, x, r, w, w_loop):
    return pl.pallas_call(
        _layer_common,
        grid=(N_NODES // ROW_BLK,),
        in_specs=_LAYER_SPECS,
        out_specs=pl.BlockSpec((ROW_BLK, DIM), lambda i: (i, 0)),
        out_shape=jax.ShapeDtypeStruct((N_NODES, DIM), jnp.float32),
    )(acc, c, x, r, w, w_loop)


def kernel(edge_index, edge_type, sub, rel, init_embed, init_rel,
           W1, W1_loop, Wrel1, W2, W2_loop, Wrel2):
    src = edge_index[0]
    dst = edge_index[1]
    e = src.shape[0]
    pad = E_PAD - e
    ar = jnp.arange(pad, dtype=jnp.int32)
    src_b, dst_b = lax.optimization_barrier((src, dst))
    srcp = jnp.concatenate([src_b, (ar * 29) % N_NODES])
    dstp = jnp.concatenate([dst_b, N_NODES + (ar % 16)])

    cf = _hist_call(dst, edge_type)
    c = cf.reshape(C_ROWS, N_RELS)

    a1 = _spmm_call(init_embed, srcp, dstp)
    x1, r1, r2 = _layer1_call(a1, c, init_embed, init_rel,
                              W1, W1_loop, Wrel1, Wrel2)
    a2 = _spmm_call(x1, srcp, dstp)
    x2 = _layer2_call(a2, c, x1, r1, W2, W2_loop)

    sub_emb, rel_emb = _gather_call(x2, sub, r2, rel)
    return (sub_emb, rel_emb, x2, r2)


# speedup vs baseline: 1.0817x; 1.0654x over previous
import functools

import jax
import jax.numpy as jnp
from jax import lax
from jax.experimental import pallas as pl
from jax.experimental.pallas import tpu as pltpu
from jax.experimental.pallas import tpu_sc as plsc

N_NODES = 10000
N_RELS = 200
DIM = 128
NW = 32
CHUNK = 128
CPW = 80
E_PAD = NW * CPW * CHUNK
ACC_ROWS = 10112
C_ROWS = 10080
CFLAT = C_ROWS * N_RELS
ROW_BLK = 2000

_mesh = plsc.VectorSubcoreMesh(core_axis_name="c", subcore_axis_name="s")

_Z16 = functools.partial(jnp.zeros, (16,), jnp.float32)


CHALF = CFLAT // 2
CDUMP = 16000
CLOC = CHALF + CDUMP


N_E = 320000
HGRP = 312
SCH = 64
GEDGES = 1024
TILE_E = E_PAD // NW
NGRP = TILE_E // GEDGES
NCHK = TILE_E // SCH


HG = 2000


def _hist_body(dstp, etp, out, cacc, dba, dbb, idxb0, idxb1, ones, wb0, wb1,
               zbuf, sem0, sem1, ssem0, ssem1, isem):
    cid = lax.axis_index("c")
    sid = lax.axis_index("s")
    base = cid * CHALF

    @pl.loop(0, 500)
    def _zfill(i):
        zbuf[pl.ds(i * 16, 16)] = _Z16()

    @pl.loop(0, 8)
    def _ofill(i):
        ones[pl.ds(i * 16, 16)] = jnp.ones((16,), jnp.float32)

    @pl.loop(0, 8)
    def _zero(i):
        pltpu.sync_copy(zbuf, cacc.at[pl.ds(sid * (CLOC // 16) + i * 8000, 8000)])

    plsc.subcore_barrier()

    def chunk_idx(db, j, idxb):
        for k in range(CHUNK // 16):
            d = db[0][pl.ds(j * 125 + k * 16, 16)]
            t = db[1][pl.ds(j * 125 + k * 16, 16)]
            loc = d * N_RELS + t - base
            lane = lax.iota(jnp.int32, 16)
            ok = jnp.logical_and(loc >= 0, loc < CHALF)
            if k == 7:
                ok = jnp.logical_and(ok, lane < 13)
            dump = CHALF + k * 16 + lane
            idxb[pl.ds(k * 16, 16)] = jnp.where(ok, loc, dump)

    def swait(idxb, ssem):
        pltpu.make_async_copy(ones, cacc.at[idxb], ssem).wait()

    def stage(g, db):
        for a, src in ((0, dstp), (1, etp)):
            pltpu.async_copy(src.at[pl.ds(sid * (N_E // 16) + g * HG, HG)],
                             db[a].at[pl.ds(0, HG)], isem)

    def wait_stage(db):
        for a, src in ((0, dstp), (1, etp)):
            pltpu.make_async_copy(src.at[pl.ds(0, HG)],
                                  db[a].at[pl.ds(0, HG)], isem).wait()

    def process(g0, db, i):
        for j in range(16):
            idxb = idxb0 if j % 2 == 0 else idxb1
            ssem = ssem0 if j % 2 == 0 else ssem1
            if g0 == 0 and j < 2:
                @pl.when(i > 0)
                def _w():
                    swait(idxb, ssem)
            else:
                swait(idxb, ssem)
            chunk_idx(db, j, idxb)
            pltpu.async_copy(ones, cacc.at[idxb], ssem, add=True)

    stage(0, dba)
    wait_stage(dba)
    stage(1, dbb)

    @pl.loop(0, 5)
    def _grp(i):
        process(0, dba, i)
        wait_stage(dbb)

        @pl.when(i < 4)
        def _ra():
            stage(2 * i + 2, dba)

        process(1, dbb, i)

        @pl.when(i < 4)
        def _rab():
            wait_stage(dba)
            stage(2 * i + 3, dbb)

    swait(idxb0, ssem0)
    swait(idxb1, ssem1)
    plsc.subcore_barrier()

    wout = base + sid * (CHALF // 16)
    wloc = sid * (CHALF // 16)

    def _piece(i, buf, sem):
        pltpu.sync_copy(cacc.at[pl.ds(wloc + i * 9000, 9000)], buf)
        return pltpu.async_copy(buf, out.at[pl.ds(wout + i * 9000, 9000)], sem)

    d0 = _piece(0, wb0, sem0)

    @pl.loop(0, 3)
    def _wo(i):
        d1 = _piece(2 * i + 1, wb1, sem1)
        pltpu.make_async_copy(wb0, out.at[pl.ds(wout, 9000)], sem0).wait()

        @pl.when(i < 2)
        def _more():
            _piece(2 * i + 2, wb0, sem0)

        d1.wait()

    _piece(6, wb0, sem0).wait()


_hist_call = functools.partial(
    pl.kernel,
    _hist_body,
    out_type=jax.ShapeDtypeStruct((CFLAT,), jnp.float32),
    mesh=_mesh,
    scratch_types=[
        pltpu.VMEM_SHARED((CLOC,), jnp.float32),
        [pltpu.VMEM((2048,), jnp.int32)] * 2,
        [pltpu.VMEM((2048,), jnp.int32)] * 2,
        pltpu.VMEM((CHUNK,), jnp.int32),
        pltpu.VMEM((CHUNK,), jnp.int32),
        pltpu.VMEM((CHUNK,), jnp.float32),
        pltpu.VMEM((9000,), jnp.float32),
        pltpu.VMEM((9000,), jnp.float32),
        pltpu.VMEM((8000,), jnp.float32),
        pltpu.SemaphoreType.DMA,
        pltpu.SemaphoreType.DMA,
        pltpu.SemaphoreType.DMA,
        pltpu.SemaphoreType.DMA,
        pltpu.SemaphoreType.DMA,
    ],
)()


def _spmm_body(table, srcp, dstp, out, acc, srcba, dstba, srcbb, dstbb,
               rows, dsaves, gsems, ssems, isem):
    cid = lax.axis_index("c")
    sid = lax.axis_index("s")
    wid = cid * 16 + sid
    myrow = sid * (ACC_ROWS // 16)
    ebase = wid * TILE_E

    @pl.loop(0, SCH)
    def _zfill(i):
        for k in range(DIM // 16):
            rows[0][i, pl.ds(k * 16, 16)] = _Z16()

    @pl.loop(0, 9)
    def _zero(i):
        pltpu.sync_copy(rows[0], acc.at[pl.ds(myrow + i * SCH, SCH)])

    pltpu.sync_copy(rows[0].at[pl.ds(0, 56)], acc.at[pl.ds(myrow + 576, 56)])
    plsc.subcore_barrier()

    def stage(gidx, sb, db, sync):
        s = pltpu.async_copy(srcp.at[pl.ds(ebase + gidx * GEDGES, GEDGES)],
                             sb, isem)
        d = pltpu.async_copy(dstp.at[pl.ds(ebase + gidx * GEDGES, GEDGES)],
                             db, isem)
        if sync:
            s.wait()
            d.wait()

    def wait_stage(sb, db):
        pltpu.make_async_copy(srcp.at[pl.ds(0, GEDGES)], sb, isem).wait()
        pltpu.make_async_copy(dstp.at[pl.ds(0, GEDGES)], db, isem).wait()

    def gissue(b, sb, off):
        pltpu.async_copy(table.at[sb.at[pl.ds(off * SCH, SCH)]],
                         rows[b], gsems[b])

    def gwait(b, sb):
        pltpu.make_async_copy(table.at[sb.at[pl.ds(0, SCH)]],
                              rows[b], gsems[b]).wait()

    def sissue(b):
        pltpu.async_copy(rows[b], acc.at[dsaves[b]], ssems[b], add=True)

    def swait(b):
        pltpu.make_async_copy(rows[b], acc.at[dsaves[b]], ssems[b]).wait()

    stage(0, srcba, dstba, True)
    stage(1, srcbb, dstbb, False)
    for c in range(3):
        gissue(c, srcba, c)

    last = NCHK // 32 - 1

    @pl.loop(0, NCHK // 32)
    def _edges(i):
        for local in range(32):
            b = local % 4
            sb, db = (srcba, dstba) if local < 16 else (srcbb, dstbb)
            off = local % 16
            gwait(b, sb)
            for k in range(SCH // 16):
                dsaves[b][pl.ds(k * 16, 16)] = db[pl.ds(off * SCH + k * 16, 16)]
            sissue(b)
            pb = (b + 3) % 4
            if local == 0:
                @pl.when(i > 0)
                def _w0():
                    swait(pb)

                gissue(pb, sb, off + 3)
            elif local == 13:
                swait(pb)
                wait_stage(srcbb, dstbb)
                gissue(pb, srcbb, 0)
            elif local < 13:
                swait(pb)
                gissue(pb, sb, off + 3)
            elif local < 29:
                swait(pb)
                gissue(pb, srcbb, local - 13)
                if local == 15:
                    @pl.when(i < last)
                    def _ra():
                        stage(2 * i + 2, srcba, dstba, False)
            elif local == 29:
                @pl.when(i < last)
                def _l29():
                    swait(pb)
                    wait_stage(srcba, dstba)
                    gissue(pb, srcba, 0)
            else:
                @pl.when(i < last)
                def _gt():
                    swait(pb)
                    gissue(pb, srcba, local - 29)

                if local == 31:
                    @pl.when(i < last)
                    def _rb():
                        stage(2 * i + 3, srcbb, dstbb, False)

    for b in range(4):
        swait(b)
    plsc.subcore_barrier()

    def _wo(b, p, n):
        return pltpu.make_async_copy(
            rows[b].at[pl.ds(0, n)],
            out.at[cid, pl.ds(myrow + p * SCH, n)], ssems[b])

    sizes = [SCH] * 9 + [56]
    for p, n in enumerate(sizes):
        b = p % 4
        if p >= 4:
            _wo(b, p - 4, sizes[p - 4]).wait()
        pltpu.sync_copy(acc.at[pl.ds(myrow + p * SCH, n)],
                        rows[b].at[pl.ds(0, n)])
        pltpu.async_copy(rows[b].at[pl.ds(0, n)],
                         out.at[cid, pl.ds(myrow + p * SCH, n)], ssems[b])
    for p in range(6, 10):
        _wo(p % 4, p, sizes[p]).wait()


def _spmm_call(table, srcp, dstp):
    return pl.kernel(
        _spmm_body,
        out_type=jax.ShapeDtypeStruct((2, ACC_ROWS, DIM), jnp.float32),
        mesh=_mesh,
        scratch_types=[
            pltpu.VMEM_SHARED((ACC_ROWS, DIM), jnp.float32),
            pltpu.VMEM((GEDGES,), jnp.int32),
            pltpu.VMEM((GEDGES,), jnp.int32),
            pltpu.VMEM((GEDGES,), jnp.int32),
            pltpu.VMEM((GEDGES,), jnp.int32),
            [pltpu.VMEM((SCH, DIM), jnp.float32)] * 4,
            [pltpu.VMEM((SCH,), jnp.int32)] * 4,
            [pltpu.SemaphoreType.DMA] * 4,
            [pltpu.SemaphoreType.DMA] * 4,
            pltpu.SemaphoreType.DMA,
        ],
    )(table, srcp, dstp)


def _gather_body(x2, sub, r2, rel, out_sub, out_rel, idxb, rows, sem):
    cid = lax.axis_index("c")
    sid = lax.axis_index("s")
    wid = cid * 16 + sid
    base = wid * 32

    pltpu.sync_copy(sub.at[pl.ds(base, 32)], idxb)
    pltpu.async_copy(x2.at[idxb], rows, sem).wait()
    pltpu.sync_copy(rows, out_sub.at[pl.ds(base, 32)])

    pltpu.sync_copy(rel.at[pl.ds(base, 32)], idxb)
    pltpu.async_copy(r2.at[idxb], rows, sem).wait()
    pltpu.sync_copy(rows, out_rel.at[pl.ds(base, 32)])


def _gather_call(x2, sub, r2, rel):
    return pl.kernel(
        _gather_body,
        out_type=(jax.ShapeDtypeStruct((1024, DIM), jnp.float32),
                  jax.ShapeDtypeStruct((1024, DIM), jnp.float32)),
        mesh=_mesh,
        scratch_types=[
            pltpu.VMEM((32,), jnp.int32),
            pltpu.VMEM((32, DIM), jnp.float32),
            pltpu.SemaphoreType.DMA,
        ],
    )(x2, sub, r2, rel)


def _layer_common(a_ref, c_ref, x_ref, r_ref, w_ref, wl_ref, o_ref):
    a = a_ref[0] + a_ref[1]
    c = c_ref[...]
    deg = jnp.sum(c, axis=1)
    agg = a - jnp.dot(c, r_ref[...], preferred_element_type=jnp.float32)
    z = jnp.dot(agg, w_ref[...], preferred_element_type=jnp.float32)
    inv = 1.0 / jnp.maximum(deg, 1.0)
    loop = jnp.dot(x_ref[...], wl_ref[...], preferred_element_type=jnp.float32)
    o_ref[...] = jnp.tanh(z * inv[:, None] + loop)


_LAYER_SPECS = [
    pl.BlockSpec((2, ROW_BLK, DIM), lambda i: (0, i, 0)),
    pl.BlockSpec((ROW_BLK, N_RELS), lambda i: (i, 0)),
    pl.BlockSpec((ROW_BLK, DIM), lambda i: (i, 0)),
    pl.BlockSpec((N_RELS, DIM), lambda i: (0, 0)),
    pl.BlockSpec((DIM, DIM), lambda i: (0, 0)),
    pl.BlockSpec((DIM, DIM), lambda i: (0, 0)),
]
_REL_SPEC = pl.BlockSpec((N_RELS, DIM), lambda i: (0, 0))


def _layer1_body(a_ref, c_ref, x_ref, r_ref, w_ref, wl_ref, wr1_ref, wr2_ref,
                 o_ref, r1_ref, r2_ref):
    _layer_common(a_ref, c_ref, x_ref, r_ref, w_ref, wl_ref, o_ref)

    @pl.when(pl.program_id(0) == 0)
    def _rels():
        r1 = jnp.dot(r_ref[...], wr1_ref[...],
                     preferred_element_type=jnp.float32)
        r1_ref[...] = r1
        r2_ref[...] = jnp.dot(r1, wr2_ref[...],
                              preferred_element_type=jnp.float32)


def _layer1_call(acc, c, x, r, w, w_loop, wr1, wr2):
    return pl.pallas_call(
        _layer1_body,
        grid=(N_NODES // ROW_BLK,),
        in_specs=_LAYER_SPECS + [pl.BlockSpec((DIM, DIM), lambda i: (0, 0)),
                                 pl.BlockSpec((DIM, DIM), lambda i: (0, 0))],
        out_specs=(pl.BlockSpec((ROW_BLK, DIM), lambda i: (i, 0)),
                   _REL_SPEC, _REL_SPEC),
        out_shape=(jax.ShapeDtypeStruct((N_NODES, DIM), jnp.float32),
                   jax.ShapeDtypeStruct((N_RELS, DIM), jnp.float32),
                   jax.ShapeDtypeStruct((N_RELS, DIM), jnp.float32)),
    )(acc, c, x, r, w, w_loop, wr1, wr2)


def _layer2_call(acc, c, x, r, w, w_loop):
    return pl.pallas_call(
        _layer_common,
        grid=(N_NODES // ROW_BLK,),
        in_specs=_LAYER_SPECS,
        out_specs=pl.BlockSpec((ROW_BLK, DIM), lambda i: (i, 0)),
        out_shape=jax.ShapeDtypeStruct((N_NODES, DIM), jnp.float32),
    )(acc, c, x, r, w, w_loop)


def kernel(edge_index, edge_type, sub, rel, init_embed, init_rel,
           W1, W1_loop, Wrel1, W2, W2_loop, Wrel2):
    src = edge_index[0]
    dst = edge_index[1]
    e = src.shape[0]
    pad = E_PAD - e
    ar = jnp.arange(pad, dtype=jnp.int32)
    src_b, dst_b = lax.optimization_barrier((src, dst))
    srcp = jnp.concatenate([src_b, (ar * 29) % N_NODES])
    dstp = jnp.concatenate([dst_b, N_NODES + (ar % 16)])

    cf = _hist_call(dst, edge_type)
    c = cf.reshape(C_ROWS, N_RELS)

    a1 = _spmm_call(init_embed, srcp, dstp)
    x1, r1, r2 = _layer1_call(a1, c, init_embed, init_rel,
                              W1, W1_loop, Wrel1, Wrel2)
    a2 = _spmm_call(x1, srcp, dstp)
    x2 = _layer2_call(a2, c, x1, r1, W2, W2_loop)

    sub_emb, rel_emb = _gather_call(x2, sub, r2, rel)
    return (sub_emb, rel_emb, x2, r2)


# speedup vs baseline: 1.0834x; 1.0016x over previous
import functools

import jax
import jax.numpy as jnp
from jax import lax
from jax.experimental import pallas as pl
from jax.experimental.pallas import tpu as pltpu
from jax.experimental.pallas import tpu_sc as plsc

N_NODES = 10000
N_RELS = 200
DIM = 128
NW = 32
CHUNK = 128
CPW = 80
E_PAD = NW * CPW * CHUNK
ACC_ROWS = 10112
C_ROWS = 10080
CFLAT = C_ROWS * N_RELS
ROW_BLK = 2000

_mesh = plsc.VectorSubcoreMesh(core_axis_name="c", subcore_axis_name="s")

_Z16 = functools.partial(jnp.zeros, (16,), jnp.float32)


CHALF = CFLAT // 2
CDUMP = 16000
CLOC = CHALF + CDUMP


N_E = 320000
SCH = 64
GEDGES = 1024
TILE_E = E_PAD // NW
NCHK = TILE_E // SCH


HG = 2000


def _hist_body(dstp, etp, out, cacc, dba, dbb, idxb0, idxb1, ones, wb0, wb1,
               zbuf, sem0, sem1, ssem0, ssem1, isem):
    cid = lax.axis_index("c")
    sid = lax.axis_index("s")
    base = cid * CHALF

    @pl.loop(0, 500)
    def _zfill(i):
        zbuf[pl.ds(i * 16, 16)] = _Z16()

    @pl.loop(0, 8)
    def _ofill(i):
        ones[pl.ds(i * 16, 16)] = jnp.ones((16,), jnp.float32)

    @pl.loop(0, 8)
    def _zero(i):
        pltpu.sync_copy(zbuf, cacc.at[pl.ds(sid * (CLOC // 16) + i * 8000, 8000)])

    plsc.subcore_barrier()

    def chunk_idx(db, j, idxb):
        for k in range(CHUNK // 16):
            d = db[0][pl.ds(j * 125 + k * 16, 16)]
            t = db[1][pl.ds(j * 125 + k * 16, 16)]
            loc = d * N_RELS + t - base
            lane = lax.iota(jnp.int32, 16)
            ok = jnp.logical_and(loc >= 0, loc < CHALF)
            if k == 7:
                ok = jnp.logical_and(ok, lane < 13)
            dump = CHALF + k * 16 + lane
            idxb[pl.ds(k * 16, 16)] = jnp.where(ok, loc, dump)

    def swait(idxb, ssem):
        pltpu.make_async_copy(ones, cacc.at[idxb], ssem).wait()

    def stage(g, db):
        for a, src in ((0, dstp), (1, etp)):
            pltpu.async_copy(src.at[pl.ds(sid * (N_E // 16) + g * HG, HG)],
                             db[a].at[pl.ds(0, HG)], isem)

    def wait_stage(db):
        for a, src in ((0, dstp), (1, etp)):
            pltpu.make_async_copy(src.at[pl.ds(0, HG)],
                                  db[a].at[pl.ds(0, HG)], isem).wait()

    def process(g0, db, i):
        for j in range(16):
            idxb = idxb0 if j % 2 == 0 else idxb1
            ssem = ssem0 if j % 2 == 0 else ssem1
            if g0 == 0 and j < 2:
                @pl.when(i > 0)
                def _w():
                    swait(idxb, ssem)
            else:
                swait(idxb, ssem)
            chunk_idx(db, j, idxb)
            pltpu.async_copy(ones, cacc.at[idxb], ssem, add=True)

    stage(0, dba)
    wait_stage(dba)
    stage(1, dbb)

    @pl.loop(0, 5)
    def _grp(i):
        process(0, dba, i)
        wait_stage(dbb)

        @pl.when(i < 4)
        def _ra():
            stage(2 * i + 2, dba)

        process(1, dbb, i)

        @pl.when(i < 4)
        def _rab():
            wait_stage(dba)
            stage(2 * i + 3, dbb)

    swait(idxb0, ssem0)
    swait(idxb1, ssem1)
    plsc.subcore_barrier()

    wout = base + sid * (CHALF // 16)
    wloc = sid * (CHALF // 16)

    def _piece(i, buf, sem):
        pltpu.sync_copy(cacc.at[pl.ds(wloc + i * 9000, 9000)], buf)
        return pltpu.async_copy(buf, out.at[pl.ds(wout + i * 9000, 9000)], sem)

    d0 = _piece(0, wb0, sem0)

    @pl.loop(0, 3)
    def _wo(i):
        d1 = _piece(2 * i + 1, wb1, sem1)
        pltpu.make_async_copy(wb0, out.at[pl.ds(wout, 9000)], sem0).wait()

        @pl.when(i < 2)
        def _more():
            _piece(2 * i + 2, wb0, sem0)

        d1.wait()

    _piece(6, wb0, sem0).wait()


_hist_call = functools.partial(
    pl.kernel,
    _hist_body,
    out_type=jax.ShapeDtypeStruct((CFLAT,), jnp.float32),
    mesh=_mesh,
    scratch_types=[
        pltpu.VMEM_SHARED((CLOC,), jnp.float32),
        [pltpu.VMEM((2048,), jnp.int32)] * 2,
        [pltpu.VMEM((2048,), jnp.int32)] * 2,
        pltpu.VMEM((CHUNK,), jnp.int32),
        pltpu.VMEM((CHUNK,), jnp.int32),
        pltpu.VMEM((CHUNK,), jnp.float32),
        pltpu.VMEM((9000,), jnp.float32),
        pltpu.VMEM((9000,), jnp.float32),
        pltpu.VMEM((8000,), jnp.float32),
        pltpu.SemaphoreType.DMA,
        pltpu.SemaphoreType.DMA,
        pltpu.SemaphoreType.DMA,
        pltpu.SemaphoreType.DMA,
        pltpu.SemaphoreType.DMA,
    ],
)()


def _spmm_body(table, srcp, dstp, out, acc, srcba, dstba, srcbb, dstbb,
               rows, dsaves, gsems, ssems, isem):
    cid = lax.axis_index("c")
    sid = lax.axis_index("s")
    wid = cid * 16 + sid
    myrow = sid * (ACC_ROWS // 16)
    ebase = wid * TILE_E

    @pl.loop(0, SCH)
    def _zfill(i):
        for k in range(DIM // 16):
            rows[0][i, pl.ds(k * 16, 16)] = _Z16()

    @pl.loop(0, 9)
    def _zero(i):
        pltpu.sync_copy(rows[0], acc.at[pl.ds(myrow + i * SCH, SCH)])

    pltpu.sync_copy(rows[0].at[pl.ds(0, 56)], acc.at[pl.ds(myrow + 576, 56)])
    plsc.subcore_barrier()

    def stage(gidx, sb, db, sync):
        s = pltpu.async_copy(srcp.at[pl.ds(ebase + gidx * GEDGES, GEDGES)],
                             sb, isem)
        d = pltpu.async_copy(dstp.at[pl.ds(ebase + gidx * GEDGES, GEDGES)],
                             db, isem)
        if sync:
            s.wait()
            d.wait()

    def wait_stage(sb, db):
        pltpu.make_async_copy(srcp.at[pl.ds(0, GEDGES)], sb, isem).wait()
        pltpu.make_async_copy(dstp.at[pl.ds(0, GEDGES)], db, isem).wait()

    def gissue(b, sb, off):
        pltpu.async_copy(table.at[sb.at[pl.ds(off * SCH, SCH)]],
                         rows[b], gsems[b])

    def gwait(b, sb):
        pltpu.make_async_copy(table.at[sb.at[pl.ds(0, SCH)]],
                              rows[b], gsems[b]).wait()

    def sissue(b):
        pltpu.async_copy(rows[b], acc.at[dsaves[b]], ssems[b], add=True)

    def swait(b):
        pltpu.make_async_copy(rows[b], acc.at[dsaves[b]], ssems[b]).wait()

    stage(0, srcba, dstba, True)
    stage(1, srcbb, dstbb, False)
    for c in range(3):
        gissue(c, srcba, c)

    last = NCHK // 32 - 1

    @pl.loop(0, NCHK // 32)
    def _edges(i):
        for local in range(32):
            b = local % 4
            sb, db = (srcba, dstba) if local < 16 else (srcbb, dstbb)
            off = local % 16
            gwait(b, sb)
            for k in range(SCH // 16):
                dsaves[b][pl.ds(k * 16, 16)] = db[pl.ds(off * SCH + k * 16, 16)]
            sissue(b)
            pb = (b + 3) % 4
            if local == 0:
                @pl.when(i > 0)
                def _w0():
                    swait(pb)

                gissue(pb, sb, off + 3)
            elif local == 13:
                swait(pb)
                wait_stage(srcbb, dstbb)
                gissue(pb, srcbb, 0)
            elif local < 13:
                swait(pb)
                gissue(pb, sb, off + 3)
            elif local < 29:
                swait(pb)
                gissue(pb, srcbb, local - 13)
                if local == 15:
                    @pl.when(i < last)
                    def _ra():
                        stage(2 * i + 2, srcba, dstba, False)
            elif local == 29:
                @pl.when(i < last)
                def _l29():
                    swait(pb)
                    wait_stage(srcba, dstba)
                    gissue(pb, srcba, 0)
            else:
                @pl.when(i < last)
                def _gt():
                    swait(pb)
                    gissue(pb, srcba, local - 29)

                if local == 31:
                    @pl.when(i < last)
                    def _rb():
                        stage(2 * i + 3, srcbb, dstbb, False)

    for b in range(4):
        swait(b)
    plsc.subcore_barrier()

    def _wo(b, p, n):
        return pltpu.make_async_copy(
            rows[b].at[pl.ds(0, n)],
            out.at[cid, pl.ds(myrow + p * SCH, n)], ssems[b])

    sizes = [SCH] * 9 + [56]
    for p, n in enumerate(sizes):
        b = p % 4
        if p >= 4:
            _wo(b, p - 4, sizes[p - 4]).wait()
        pltpu.sync_copy(acc.at[pl.ds(myrow + p * SCH, n)],
                        rows[b].at[pl.ds(0, n)])
        pltpu.async_copy(rows[b].at[pl.ds(0, n)],
                         out.at[cid, pl.ds(myrow + p * SCH, n)], ssems[b])
    for p in range(6, 10):
        _wo(p % 4, p, sizes[p]).wait()


def _spmm_call(table, srcp, dstp):
    return pl.kernel(
        _spmm_body,
        out_type=jax.ShapeDtypeStruct((2, ACC_ROWS, DIM), jnp.float32),
        mesh=_mesh,
        scratch_types=[
            pltpu.VMEM_SHARED((ACC_ROWS, DIM), jnp.float32),
            pltpu.VMEM((GEDGES,), jnp.int32),
            pltpu.VMEM((GEDGES,), jnp.int32),
            pltpu.VMEM((GEDGES,), jnp.int32),
            pltpu.VMEM((GEDGES,), jnp.int32),
            [pltpu.VMEM((SCH, DIM), jnp.float32)] * 4,
            [pltpu.VMEM((SCH,), jnp.int32)] * 4,
            [pltpu.SemaphoreType.DMA] * 4,
            [pltpu.SemaphoreType.DMA] * 4,
            pltpu.SemaphoreType.DMA,
        ],
    )(table, srcp, dstp)


def _gather_body(x2, sub, r2, rel, out_sub, out_rel, idxb, rows, sem):
    cid = lax.axis_index("c")
    sid = lax.axis_index("s")
    wid = cid * 16 + sid
    base = wid * 32

    pltpu.sync_copy(sub.at[pl.ds(base, 32)], idxb)
    pltpu.async_copy(x2.at[idxb], rows, sem).wait()
    pltpu.sync_copy(rows, out_sub.at[pl.ds(base, 32)])

    pltpu.sync_copy(rel.at[pl.ds(base, 32)], idxb)
    pltpu.async_copy(r2.at[idxb], rows, sem).wait()
    pltpu.sync_copy(rows, out_rel.at[pl.ds(base, 32)])


def _gather_call(x2, sub, r2, rel):
    return pl.kernel(
        _gather_body,
        out_type=(jax.ShapeDtypeStruct((1024, DIM), jnp.float32),
                  jax.ShapeDtypeStruct((1024, DIM), jnp.float32)),
        mesh=_mesh,
        scratch_types=[
            pltpu.VMEM((32,), jnp.int32),
            pltpu.VMEM((32, DIM), jnp.float32),
            pltpu.SemaphoreType.DMA,
        ],
    )(x2, sub, r2, rel)


def _layer_common(a_ref, c_ref, x_ref, r_ref, w_ref, wl_ref, o_ref):
    a = a_ref[0] + a_ref[1]
    c = c_ref[...]
    deg = jnp.sum(c, axis=1)
    agg = a - jnp.dot(c, r_ref[...], preferred_element_type=jnp.float32)
    z = jnp.dot(agg, w_ref[...], preferred_element_type=jnp.float32)
    inv = 1.0 / jnp.maximum(deg, 1.0)
    loop = jnp.dot(x_ref[...], wl_ref[...], preferred_element_type=jnp.float32)
    o_ref[...] = jnp.tanh(z * inv[:, None] + loop)


_LAYER_SPECS = [
    pl.BlockSpec((2, ROW_BLK, DIM), lambda i: (0, i, 0)),
    pl.BlockSpec((ROW_BLK, N_RELS), lambda i: (i, 0)),
    pl.BlockSpec((ROW_BLK, DIM), lambda i: (i, 0)),
    pl.BlockSpec((N_RELS, DIM), lambda i: (0, 0)),
    pl.BlockSpec((DIM, DIM), lambda i: (0, 0)),
    pl.BlockSpec((DIM, DIM), lambda i: (0, 0)),
]
_REL_SPEC = pl.BlockSpec((N_RELS, DIM), lambda i: (0, 0))


def _layer1_body(a_ref, c_ref, x_ref, r_ref, w_ref, wl_ref, wr1_ref, wr2_ref,
                 o_ref, r1_ref, r2_ref):
    _layer_common(a_ref, c_ref, x_ref, r_ref, w_ref, wl_ref, o_ref)

    @pl.when(pl.program_id(0) == 0)
    def _rels():
        r1 = jnp.dot(r_ref[...], wr1_ref[...],
                     preferred_element_type=jnp.float32)
        r1_ref[...] = r1
        r2_ref[...] = jnp.dot(r1, wr2_ref[...],
                              preferred_element_type=jnp.float32)


def _layer1_call(acc, c, x, r, w, w_loop, wr1, wr2):
    return pl.pallas_call(
        _layer1_body,
        grid=(N_NODES // ROW_BLK,),
        in_specs=_LAYER_SPECS + [pl.BlockSpec((DIM, DIM), lambda i: (0, 0)),
                                 pl.BlockSpec((DIM, DIM), lambda i: (0, 0))],
        out_specs=(pl.BlockSpec((ROW_BLK, DIM), lambda i: (i, 0)),
                   _REL_SPEC, _REL_SPEC),
        out_shape=(jax.ShapeDtypeStruct((N_NODES, DIM), jnp.float32),
                   jax.ShapeDtypeStruct((N_RELS, DIM), jnp.float32),
                   jax.ShapeDtypeStruct((N_RELS, DIM), jnp.float32)),
    )(acc, c, x, r, w, w_loop, wr1, wr2)


def _layer2_call(acc, c, x, r, w, w_loop):
    return pl.pallas_call(
        _layer_common,
        grid=(N_NODES // ROW_BLK,),
        in_specs=_LAYER_SPECS,
        out_specs=pl.BlockSpec((ROW_BLK, DIM), lambda i: (i, 0)),
        out_shape=jax.ShapeDtypeStruct((N_NODES, DIM), jnp.float32),
    )(acc, c, x, r, w, w_loop)


def kernel(edge_index, edge_type, sub, rel, init_embed, init_rel,
           W1, W1_loop, Wrel1, W2, W2_loop, Wrel2):
    src = edge_index[0]
    dst = edge_index[1]
    e = src.shape[0]
    pad = E_PAD - e
    ar = jnp.arange(pad, dtype=jnp.int32)
    src_b, dst_b = lax.optimization_barrier((src, dst))
    srcp = jnp.concatenate([src_b, (ar * 29) % N_NODES])
    dstp = jnp.concatenate([dst_b, N_NODES + (ar % 16)])

    cf = _hist_call(dst, edge_type)
    c = cf.reshape(C_ROWS, N_RELS)

    a1 = _spmm_call(init_embed, srcp, dstp)
    x1, r1, r2 = _layer1_call(a1, c, init_embed, init_rel,
                              W1, W1_loop, Wrel1, Wrel2)
    a2 = _spmm_call(x1, srcp, dstp)
    x2 = _layer2_call(a2, c, x1, r1, W2, W2_loop)

    sub_emb, rel_emb = _gather_call(x2, sub, r2, rel)
    return (sub_emb, rel_emb, x2, r2)
